# baseline (speedup 1.0000x reference)
import functools

import jax
import jax.numpy as jnp
from jax import lax
from jax.experimental import pallas as pl
from jax.experimental.pallas import tpu as pltpu
from jax.experimental.pallas import tpu_sc as plsc

B = 128
C = 100000
D = 128

BC = 2048
NB = (C + BC - 1) // BC

NCORE = 2
NSUB = 16
NW = NCORE * NSUB
DPW = D // NW


def _stats_body(x_ref, m_ref, s_ref, a_ref):
    bid = pl.program_id(0)

    @pl.when(bid == 0)
    def _init():
        m_ref[...] = jnp.full((B, 1), -jnp.inf, jnp.float32)
        s_ref[...] = jnp.zeros((B, 1), jnp.float32)
        a_ref[...] = jnp.zeros((B, 1), jnp.int32)

    x = x_ref[...]
    gcol = bid * BC + lax.broadcasted_iota(jnp.int32, (1, BC), 1)
    xm = jnp.where(gcol < C, x, -jnp.inf)

    m_old = m_ref[...]
    bm = jnp.max(xm, axis=1, keepdims=True)
    m_new = jnp.maximum(m_old, bm)
    e = jnp.exp(xm - m_new)
    s_ref[...] = s_ref[...] * jnp.exp(m_old - m_new) + jnp.sum(
        e, axis=1, keepdims=True)
    m_ref[...] = m_new

    cand = jnp.min(jnp.where(xm == bm, gcol, jnp.int32(2**30)),
                   axis=1, keepdims=True)
    a_ref[...] = jnp.where(bm > m_old, cand, a_ref[...])


_stats = pl.pallas_call(
    _stats_body,
    grid=(NB,),
    in_specs=[pl.BlockSpec((B, BC), lambda i: (0, i))],
    out_specs=[
        pl.BlockSpec((B, 1), lambda i: (0, 0)),
        pl.BlockSpec((B, 1), lambda i: (0, 0)),
        pl.BlockSpec((B, 1), lambda i: (0, 0)),
    ],
    out_shape=[
        jax.ShapeDtypeStruct((B, 1), jnp.float32),
        jax.ShapeDtypeStruct((B, 1), jnp.float32),
        jax.ShapeDtypeStruct((B, 1), jnp.int32),
    ],
    compiler_params=pltpu.CompilerParams(
        dimension_semantics=("arbitrary",)),
)


def _margin_body(wf_hbm, ypf_hbm, pred_hbm, ytr_hbm, mp_hbm, t_hbm,
                 predv, ytrv, idxv, wiv, wjv, accv, bufv, tv, shared, sem):
    c = lax.axis_index("c")
    s = lax.axis_index("s")
    g = c * NSUB + s

    pltpu.sync_copy(pred_hbm, predv)
    pltpu.sync_copy(ytr_hbm, ytrv)

    @pl.when(g == 0)
    def _tgt():
        for k in range(B // 16):
            idxv[pl.ds(16 * k, 16)] = (
                (lax.iota(jnp.int32, 16) + (16 * k)) * C
                + ytrv[pl.ds(16 * k, 16)])
        pltpu.async_copy(ypf_hbm.at[idxv], tv, sem).wait()
        pltpu.sync_copy(tv, t_hbm)

    for k in range(B // 16):
        accv[pl.ds(16 * k, 16)] = jnp.zeros((16,), jnp.float32)

    for dd in range(DPW):
        off = (g * DPW + dd) * C
        for k in range(B // 16):
            idxv[pl.ds(16 * k, 16)] = predv[pl.ds(16 * k, 16)] + off
        pltpu.async_copy(wf_hbm.at[idxv], wiv, sem).wait()
        for k in range(B // 16):
            idxv[pl.ds(16 * k, 16)] = ytrv[pl.ds(16 * k, 16)] + off
        pltpu.async_copy(wf_hbm.at[idxv], wjv, sem).wait()
        for k in range(B // 16):
            sl = pl.ds(16 * k, 16)
            accv[sl] = accv[sl] + wiv[sl] * wjv[sl]

    pltpu.sync_copy(accv, shared.at[s])
    plsc.subcore_barrier()

    @pl.when(s == 0)
    def _reduce():
        pltpu.sync_copy(shared, bufv)
        for k in range(B // 16):
            sl = pl.ds(16 * k, 16)
            acc = bufv[0, sl]
            for j in range(1, NSUB):
                acc = acc + bufv[j, sl]
            accv[sl] = acc
        pltpu.sync_copy(accv, mp_hbm.at[c])


@functools.cache
def _margin_kernel():
    return pl.kernel(
        _margin_body,
        out_type=[
            jax.ShapeDtypeStruct((NCORE, B), jnp.float32),
            jax.ShapeDtypeStruct((B,), jnp.float32),
        ],
        mesh=plsc.VectorSubcoreMesh(
            core_axis_name="c", subcore_axis_name="s",
            num_cores=NCORE, num_subcores=NSUB),
        scratch_types=[
            pltpu.VMEM((B,), jnp.int32),
            pltpu.VMEM((B,), jnp.int32),
            pltpu.VMEM((B,), jnp.int32),
            pltpu.VMEM((B,), jnp.float32),
            pltpu.VMEM((B,), jnp.float32),
            pltpu.VMEM((B,), jnp.float32),
            pltpu.VMEM((NSUB, B), jnp.float32),
            pltpu.VMEM((B,), jnp.float32),
            pltpu.VMEM_SHARED((NSUB, B), jnp.float32),
            pltpu.SemaphoreType.DMA,
        ],
    )


def _loss_body(m_ref, s_ref, t_ref, mp_ref, o_ref):
    m = m_ref[...]
    sm = s_ref[...]
    t = t_ref[...]
    margin = mp_ref[0:1, :] + mp_ref[1:2, :]
    zz = sm - jnp.exp(t - m) + jnp.exp(t - margin - m)
    logz = m + jnp.log(zz)
    o_ref[...] = jnp.sum(logz - t + margin, axis=1, keepdims=True) * (1.0 / B)


_loss = pl.pallas_call(
    _loss_body,
    out_shape=jax.ShapeDtypeStruct((1, 1), jnp.float32),
)


@jax.jit
def kernel(y_pred, y_true, W):
    y_true = y_true.astype(jnp.int32)
    m, s, a = _stats(y_pred)
    mp, t = _margin_kernel()(W.reshape(D * C), y_pred.reshape(B * C),
                             a.reshape(B), y_true)
    loss = _loss(m.reshape(1, B), s.reshape(1, B), t.reshape(1, B), mp)
    return loss.reshape(())


# speedup vs baseline: 1.4302x; 1.4302x over previous
import jax
import jax.numpy as jnp
from jax import lax
from jax.experimental import pallas as pl
from jax.experimental.pallas import tpu as pltpu

B = 128
C = 100000
D = 128

BC = 8192
NB = (C + BC - 1) // BC

BW = 128


def _stats_body(yt_ref, x_ref, m_ref, s_ref, a_ref, t_ref):
    bid = pl.program_id(0)

    @pl.when(bid == 0)
    def _init():
        m_ref[...] = jnp.full((B, 1), -jnp.inf, jnp.float32)
        s_ref[...] = jnp.zeros((B, 1), jnp.float32)
        a_ref[...] = jnp.zeros((B, 1), jnp.int32)
        t_ref[...] = jnp.zeros((B, 1), jnp.float32)

    x = x_ref[...]
    gcol = bid * BC + lax.broadcasted_iota(jnp.int32, (1, BC), 1)
    xm = jnp.where(gcol < C, x, -jnp.inf)

    m_old = m_ref[...]
    bm = jnp.max(xm, axis=1, keepdims=True)
    m_new = jnp.maximum(m_old, bm)
    e = jnp.exp(xm - m_new)
    s_ref[...] = s_ref[...] * jnp.exp(m_old - m_new) + jnp.sum(
        e, axis=1, keepdims=True)
    m_ref[...] = m_new

    cand = jnp.min(jnp.where(xm == bm, gcol, jnp.int32(2**30)),
                   axis=1, keepdims=True)
    a_ref[...] = jnp.where(bm > m_old, cand, a_ref[...])

    t_ref[...] = t_ref[...] + jnp.sum(
        jnp.where(gcol == yt_ref[...], x, 0.0), axis=1, keepdims=True)


_stats = pl.pallas_call(
    _stats_body,
    grid=(NB,),
    in_specs=[
        pl.BlockSpec((B, 1), lambda i: (0, 0)),
        pl.BlockSpec((B, BC), lambda i: (0, i)),
    ],
    out_specs=[
        pl.BlockSpec((B, 1), lambda i: (0, 0)),
        pl.BlockSpec((B, 1), lambda i: (0, 0)),
        pl.BlockSpec((B, 1), lambda i: (0, 0)),
        pl.BlockSpec((B, 1), lambda i: (0, 0)),
    ],
    out_shape=[
        jax.ShapeDtypeStruct((B, 1), jnp.float32),
        jax.ShapeDtypeStruct((B, 1), jnp.float32),
        jax.ShapeDtypeStruct((B, 1), jnp.int32),
        jax.ShapeDtypeStruct((B, 1), jnp.float32),
    ],
    compiler_params=pltpu.CompilerParams(
        dimension_semantics=("arbitrary",)),
)


def _margin_body(idx_ref, wp_ref, wt_ref, m_ref, s_ref, t_ref, o_ref, acc):
    i = pl.program_id(0)

    @pl.when(i == 0)
    def _init():
        acc[...] = jnp.zeros((1, B), jnp.float32)

    lane = lax.broadcasted_iota(jnp.int32, (1, BW), 1)
    jp = idx_ref[0, i] % BW
    jt = idx_ref[1, i] % BW
    wi = jnp.sum(jnp.where(lane == jp, wp_ref[...], 0.0),
                 axis=1, keepdims=True)
    wj = jnp.sum(jnp.where(lane == jt, wt_ref[...], 0.0),
                 axis=1, keepdims=True)
    mg = jnp.sum(wi * wj, axis=0, keepdims=True)
    lane_b = lax.broadcasted_iota(jnp.int32, (1, B), 1)
    margin = acc[...] + jnp.where(lane_b == i, mg, 0.0)
    acc[...] = margin

    @pl.when(i == B - 1)
    def _loss():
        m = m_ref[...]
        sm = s_ref[...]
        t = t_ref[...]
        zz = sm - jnp.exp(t - m) + jnp.exp(t - margin - m)
        logz = m + jnp.log(zz)
        o_ref[...] = jnp.sum(logz - t + margin, axis=1,
                             keepdims=True) * (1.0 / B)


_margin = pl.pallas_call(
    _margin_body,
    grid_spec=pltpu.PrefetchScalarGridSpec(
        num_scalar_prefetch=1,
        grid=(B,),
        in_specs=[
            pl.BlockSpec((D, BW), lambda i, idx: (0, idx[0, i] // BW)),
            pl.BlockSpec((D, BW), lambda i, idx: (0, idx[1, i] // BW)),
            pl.BlockSpec((1, B), lambda i, idx: (0, 0)),
            pl.BlockSpec((1, B), lambda i, idx: (0, 0)),
            pl.BlockSpec((1, B), lambda i, idx: (0, 0)),
        ],
        out_specs=pl.BlockSpec((1, 1), lambda i, idx: (0, 0)),
        scratch_shapes=[pltpu.VMEM((1, B), jnp.float32)],
    ),
    out_shape=jax.ShapeDtypeStruct((1, 1), jnp.float32),
    compiler_params=pltpu.CompilerParams(
        dimension_semantics=("arbitrary",)),
)


@jax.jit
def kernel(y_pred, y_true, W):
    y_true = y_true.astype(jnp.int32)
    m, s, a, t = _stats(y_true.reshape(B, 1), y_pred)
    idx = jnp.stack([a.reshape(B), y_true])
    loss = _margin(idx, W, W, m.reshape(1, B), s.reshape(1, B),
                   t.reshape(1, B))
    return loss.reshape(())


# speedup vs baseline: 1.9041x; 1.3314x over previous
import jax
import jax.numpy as jnp
from jax import lax
from jax.experimental import pallas as pl
from jax.experimental.pallas import tpu as pltpu

B = 128
C = 100000
D = 128

BC = 8192
NB = (C + BC - 1) // BC


def _fused_body(ytc_ref, ytr_ref, x_ref, w_ref, o_ref,
                m_s, s_s, a_s, t_s, pr_s, wi_s, wj_s):
    i = pl.program_id(0)

    @pl.when(i == 0)
    def _init():
        m_s[...] = jnp.full((B, 1), -jnp.inf, jnp.float32)
        s_s[...] = jnp.zeros((B, 1), jnp.float32)
        a_s[...] = jnp.zeros((B, 1), jnp.int32)
        t_s[...] = jnp.zeros((B, 1), jnp.float32)
        wi_s[...] = jnp.zeros((D, B), jnp.float32)
        wj_s[...] = jnp.zeros((D, B), jnp.float32)

    @pl.when(i < NB)
    def _phase_a():
        x = x_ref[...]
        gcol = i * BC + lax.broadcasted_iota(jnp.int32, (1, BC), 1)
        xm = jnp.where(gcol < C, x, -jnp.inf)

        m_old = m_s[...]
        bm = jnp.max(xm, axis=1, keepdims=True)
        m_new = jnp.maximum(m_old, bm)
        s_s[...] = s_s[...] * jnp.exp(m_old - m_new) + jnp.sum(
            jnp.exp(xm - m_new), axis=1, keepdims=True)
        m_s[...] = m_new

        cand = jnp.min(jnp.where(xm == bm, gcol, jnp.int32(2**30)),
                       axis=1, keepdims=True)
        a_s[...] = jnp.where(bm > m_old, cand, a_s[...])

        t_s[...] = t_s[...] + jnp.sum(
            jnp.where(gcol == ytc_ref[...], x, 0.0), axis=1, keepdims=True)

    @pl.when(i == NB)
    def _pred_to_row():
        eye = (lax.broadcasted_iota(jnp.int32, (B, B), 0) ==
               lax.broadcasted_iota(jnp.int32, (B, B), 1)).astype(jnp.float32)
        pr_s[...] = lax.dot_general(
            a_s[...].astype(jnp.float32), eye, (((0,), (0,)), ((), ())),
            preferred_element_type=jnp.float32)

    @pl.when(i >= NB)
    def _phase_b():
        j = i - NB
        gcol_c = j * BC + lax.broadcasted_iota(jnp.int32, (BC, 1), 0)
        gcol_r = j * BC + lax.broadcasted_iota(jnp.int32, (1, BC), 1)
        gcolf = gcol_c.astype(jnp.float32)
        ohp = (gcolf == pr_s[...]).astype(jnp.bfloat16)
        oht = (gcolf == ytr_ref[...]).astype(jnp.bfloat16)
        wb = jnp.where(gcol_r < C, w_ref[...], 0.0).astype(jnp.bfloat16)
        wi_s[...] = wi_s[...] + lax.dot_general(
            wb, ohp, (((1,), (0,)), ((), ())),
            preferred_element_type=jnp.float32)
        wj_s[...] = wj_s[...] + lax.dot_general(
            wb, oht, (((1,), (0,)), ((), ())),
            preferred_element_type=jnp.float32)

    @pl.when(i == 2 * NB - 1)
    def _loss():
        eye = (lax.broadcasted_iota(jnp.int32, (B, B), 0) ==
               lax.broadcasted_iota(jnp.int32, (B, B), 1)).astype(jnp.float32)
        mrow = jnp.sum(wi_s[...] * wj_s[...], axis=0, keepdims=True)
        mcol = lax.dot_general(eye, mrow, (((1,), (1,)), ((), ())),
                               preferred_element_type=jnp.float32)
        m = m_s[...]
        t = t_s[...]
        zz = s_s[...] - jnp.exp(t - m) + jnp.exp(t - mcol - m)
        lossv = m + jnp.log(zz) - t + mcol
        o_ref[...] = jnp.sum(lossv, axis=0, keepdims=True) * (1.0 / B)


_fused = pl.pallas_call(
    _fused_body,
    grid=(2 * NB,),
    in_specs=[
        pl.BlockSpec((B, 1), lambda i: (0, 0)),
        pl.BlockSpec((1, B), lambda i: (0, 0)),
        pl.BlockSpec((B, BC), lambda i: (0, jnp.minimum(i, NB - 1))),
        pl.BlockSpec((D, BC), lambda i: (0, jnp.maximum(i - NB, 0))),
    ],
    out_specs=pl.BlockSpec((1, 1), lambda i: (0, 0)),
    out_shape=jax.ShapeDtypeStruct((1, 1), jnp.float32),
    scratch_shapes=[
        pltpu.VMEM((B, 1), jnp.float32),
        pltpu.VMEM((B, 1), jnp.float32),
        pltpu.VMEM((B, 1), jnp.int32),
        pltpu.VMEM((B, 1), jnp.float32),
        pltpu.VMEM((1, B), jnp.float32),
        pltpu.VMEM((D, B), jnp.float32),
        pltpu.VMEM((D, B), jnp.float32),
    ],
    compiler_params=pltpu.CompilerParams(
        dimension_semantics=("arbitrary",)),
)


@jax.jit
def kernel(y_pred, y_true, W):
    y_true = y_true.astype(jnp.int32)
    loss = _fused(y_true.reshape(B, 1),
                  y_true.astype(jnp.float32).reshape(1, B),
                  y_pred, W)
    return loss.reshape(())
